# Initial kernel scaffold; baseline (speedup 1.0000x reference)
import jax
import jax.numpy as jnp
from jax.experimental import pallas as pl


def kernel(x, params):
    raise NotImplementedError("write your pallas kernel here")


# speedup vs baseline: 1.2806x; 1.2806x over previous
import functools
import math

import jax
import jax.numpy as jnp
from jax.experimental import pallas as pl

E, K, H = 20, 3, 3


def _gate_kernel(x_ref, wq_ref, bq_ref, wk_ref, bk_ref, xg_ref, *, S, D):
    x = x_ref[0]
    q = jnp.dot(x, wq_ref[...], preferred_element_type=jnp.float32) + bq_ref[...]
    k = jnp.dot(x, wk_ref[...], preferred_element_type=jnp.float32) + bk_ref[...]
    a = None
    for h in range(H):
        qh = jnp.transpose(q[:, E * h:E * (h + 1)])
        kh = jnp.transpose(k[:, E * h:E * (h + 1)])
        term = qh[:, :, None] * kh[:, None, :]
        a = term if a is None else a + term
    a = a * (1.0 / math.sqrt(H))
    m = jnp.max(a, axis=2, keepdims=True)
    z = jnp.exp(a - m)
    s = jnp.sum(z, axis=2, keepdims=True)
    gate = jnp.sum(z / s, axis=1)
    iota = jax.lax.broadcasted_iota(jnp.int32, (E, S), 1)
    score = gate
    toks = []
    for _ in range(K):
        mx = jnp.max(score, axis=1, keepdims=True)
        cand = jnp.where(score == mx, iota, S)
        idx = jnp.min(cand, axis=1, keepdims=True)
        onehot = iota == idx
        sel = jnp.where(onehot, mx, 0.0)
        toks.append(jnp.dot(sel, x, preferred_element_type=jnp.float32))
        score = jnp.where(onehot, -jnp.inf, score)
    xg_ref[0] = jnp.concatenate(toks, axis=1)


def _gate_call(x, p):
    B, S, D = x.shape
    kern = functools.partial(_gate_kernel, S=S, D=D)
    return pl.pallas_call(
        kern,
        grid=(B,),
        in_specs=[
            pl.BlockSpec((1, S, D), lambda b: (b, 0, 0)),
            pl.BlockSpec((D, H * E), lambda b: (0, 0)),
            pl.BlockSpec((1, H * E), lambda b: (0, 0)),
            pl.BlockSpec((D, H * E), lambda b: (0, 0)),
            pl.BlockSpec((1, H * E), lambda b: (0, 0)),
        ],
        out_specs=pl.BlockSpec((1, E, K * D), lambda b: (b, 0, 0)),
        out_shape=jax.ShapeDtypeStruct((B, E, K * D), jnp.float32),
    )(x, p['Wq'], p['bq'].reshape(1, -1), p['Wk'], p['bk'].reshape(1, -1))


def _mlp_kernel(xg_ref, w1_ref, b1_ref, w2_ref, b2_ref, w3_ref, b3_ref,
                o_ref, *, act):
    for e in range(E):
        xg = xg_ref[:, e, :]
        h = jnp.dot(xg, w1_ref[e], preferred_element_type=jnp.float32)
        h = jnp.maximum(h + b1_ref[e:e + 1, :], 0.0)
        h = jnp.dot(h, w2_ref[e], preferred_element_type=jnp.float32)
        h = jnp.maximum(h + b2_ref[e:e + 1, :], 0.0)
        o = jnp.dot(h, w3_ref[e], preferred_element_type=jnp.float32)
        o = o + b3_ref[e:e + 1, :]
        if act == 'sigmoid':
            o = jax.nn.sigmoid(o)
        else:
            o = jnp.maximum(o, 0.0)
        o_ref[:, e, :] = o


def _mlp_call(xg, p, act):
    B, _, KD = xg.shape
    dout = p['W1'].shape[-1]
    kern = functools.partial(_mlp_kernel, act=act)
    return pl.pallas_call(
        kern,
        out_shape=jax.ShapeDtypeStruct((B, E, dout), jnp.float32),
    )(xg, p['W1'], p['b1'], p['W2'], p['b2'], p['W3'], p['b3'])


def _fc_kernel(x_ref, w_ref, b_ref, o_ref):
    o = jnp.dot(x_ref[...], w_ref[...], preferred_element_type=jnp.float32)
    o_ref[...] = jnp.maximum(o + b_ref[...], 0.0)


def _fc_call(x, w, b, tile=640):
    B, Din = x.shape
    Dout = w.shape[1]
    return pl.pallas_call(
        _fc_kernel,
        grid=(Dout // tile,),
        in_specs=[
            pl.BlockSpec((B, Din), lambda n: (0, 0)),
            pl.BlockSpec((Din, tile), lambda n: (0, n)),
            pl.BlockSpec((1, tile), lambda n: (0, n)),
        ],
        out_specs=pl.BlockSpec((B, tile), lambda n: (0, n)),
        out_shape=jax.ShapeDtypeStruct((B, Dout), jnp.float32),
    )(x, w, b.reshape(1, -1))


def _last_kernel(x_ref, w1_ref, b1_ref, w2_ref, b2_ref, o_ref):
    h = jnp.dot(x_ref[...], w1_ref[...], preferred_element_type=jnp.float32)
    h = h + b1_ref[...]
    o = jnp.dot(h, w2_ref[...], preferred_element_type=jnp.float32)
    o_ref[...] = o + b2_ref[...]


def _last_call(x, w1, b1, w2, b2):
    B = x.shape[0]
    return pl.pallas_call(
        _last_kernel,
        out_shape=jax.ShapeDtypeStruct((B, w2.shape[1]), jnp.float32),
    )(x, w1, b1.reshape(1, -1), w2, b2.reshape(1, -1))


def kernel(x, params):
    B = x.shape[0]
    x = x.reshape(B, x.shape[1], -1)

    xg = _gate_call(x, params['moe1'])
    y = _mlp_call(xg, params['moe1'], act='sigmoid')
    y = _fc_call(y.reshape(B, -1), params['fc1_W'], params['fc1_b'])

    xg = _gate_call(y.reshape(B, E, 128), params['moe2'])
    y = _mlp_call(xg, params['moe2'], act='relu')
    y = _fc_call(y.reshape(B, -1), params['fc2_W'], params['fc2_b'])

    xg = _gate_call(y.reshape(B, E, 128), params['moe3'])
    y = _mlp_call(xg, params['moe3'], act='sigmoid')
    y = _fc_call(y.reshape(B, -1), params['fc3_W'], params['fc3_b'])

    return _last_call(y.reshape(B, -1), params['last_W'], params['last_b'],
                      params['last2_W'], params['last2_b'])
